# baseline (speedup 1.0000x reference)
import math

import jax
import jax.numpy as jnp
import numpy as np
from jax.experimental import pallas as pl
from jax.experimental.pallas import tpu as pltpu

_L = 2048
_H = 16
_DH = 64
_P = _H // 2
_NODES = 8
_U = min(5 * int(math.ceil(math.log(_L + 1))), _L)

def _count_matrix():
    idx = jax.random.randint(jax.random.key(42), (_L, _U), 0, _L)
    cnt = jnp.zeros((_L, _L), jnp.int8)
    rows = jnp.broadcast_to(jnp.arange(_L, dtype=idx.dtype)[:, None], idx.shape)
    return cnt.at[rows, idx].add(1)

_BLK = 256
_DN = (((1,), (1,)), ((), ()))


def _qkv_proj(x_q, x_k, x_v, wq, bq, wk, bk, wv, bv, scale):
    c = x_q.shape[1]

    def body(xq_ref, xk_ref, xv_ref, wq_ref, bq_ref, wk_ref, bk_ref,
             wv_ref, bv_ref, q_ref, k_ref, v_ref):
        yq = (jax.lax.dot_general(xq_ref[...], wq_ref[...], _DN,
                                  preferred_element_type=jnp.float32)
              + bq_ref[...]) * scale
        yk = jax.lax.dot_general(xk_ref[...], wk_ref[...], _DN,
                                 preferred_element_type=jnp.float32) + bk_ref[...]
        yv = jax.lax.dot_general(xv_ref[...], wv_ref[...], _DN,
                                 preferred_element_type=jnp.float32) + bv_ref[...]
        for p in range(_P):
            sl = slice(p * 128, (p + 1) * 128)
            q_ref[p] = yq[:, sl]
            k_ref[p] = yk[:, sl]
            v_ref[p] = yv[:, sl]

    full_w = pl.BlockSpec((c, c), lambda i: (0, 0))
    full_b = pl.BlockSpec((1, c), lambda i: (0, 0))
    xspec = pl.BlockSpec((_BLK, c), lambda i: (i, 0))
    ospec = pl.BlockSpec((_P, _BLK, 2 * _DH), lambda i: (0, i, 0))
    oshape = jax.ShapeDtypeStruct((_P, _L, 2 * _DH), jnp.float32)
    return pl.pallas_call(
        body,
        grid=(_L // _BLK,),
        in_specs=[xspec, xspec, xspec, full_w, full_b, full_w, full_b,
                  full_w, full_b],
        out_specs=[ospec, ospec, ospec],
        out_shape=[oshape, oshape, oshape],
    )(x_q, x_k, x_v, wq, bq.reshape(1, c), wk, bk.reshape(1, c),
      wv, bv.reshape(1, c))


def _meas(q3, k3, cnt):
    def body(q_ref, k_ref, c_ref, o_ref):
        p = pl.program_id(1)
        cf = c_ref[...].astype(jnp.float32)
        kp = k_ref[p]
        for s in range(2):
            sl = slice(s * _DH, (s + 1) * _DH)
            qk = jax.lax.dot_general(
                q_ref[0][:, sl], kp[:, sl], _DN,
                preferred_element_type=jnp.float32)
            mx = jnp.max(jnp.where(cf > 0.0, qk, -jnp.inf), axis=1)
            sm = jnp.sum(qk * cf, axis=1) * (1.0 / _U)
            o_ref[0, s, :] = mx - sm

    return pl.pallas_call(
        body,
        grid=(_L // _BLK, _P),
        in_specs=[
            pl.BlockSpec((1, _BLK, 2 * _DH), lambda i, p: (p, i, 0)),
            pl.BlockSpec((_P, _L, 2 * _DH), lambda i, p: (0, 0, 0)),
            pl.BlockSpec((_BLK, _L), lambda i, p: (i, 0)),
        ],
        out_specs=pl.BlockSpec((1, 2, _BLK), lambda i, p: (p, 0, i)),
        out_shape=jax.ShapeDtypeStruct((_P, 2, _L), jnp.float32),
    )(q3, k3, cnt)


def _attend(meas, q3, k3, v3):
    def body(m_ref, q_ref, k_ref, v_ref, o_ref, oh_ref, ms_ref):
        iota_l = jax.lax.broadcasted_iota(jnp.int32, (1, _L), 1)
        mv = jnp.concatenate([m_ref[:, 0, :], m_ref[:, 1, :]], axis=0)
        for t in range(_U):
            m = jnp.max(mv, axis=1, keepdims=True)
            isel = jnp.min(jnp.where(mv == m, iota_l, _L), axis=1,
                           keepdims=True)
            hit = iota_l == isel
            oh_ref[:, t, :] = hit.astype(jnp.float32)
            ms_ref[:, t, :] = (
                (iota_l // _NODES) <= (isel // _NODES)).astype(jnp.float32)
            mv = jnp.where(hit, -jnp.inf, mv)

        for p in range(_P):
            ctx = []
            for s in range(2):
                r = p + s * _P
                sl = slice(s * _DH, (s + 1) * _DH)
                oh = oh_ref[r]
                qh = q_ref[p][:, sl]
                kh = k_ref[p][:, sl]
                vh = v_ref[p][:, sl]
                q_red = jax.lax.dot_general(
                    oh, qh, (((1,), (0,)), ((), ())),
                    preferred_element_type=jnp.float32)
                scores = jax.lax.dot_general(
                    q_red, kh, _DN,
                    preferred_element_type=jnp.float32)
                scores = jnp.where(ms_ref[r] > 0.5, scores, -jnp.inf)
                smax = jnp.max(scores, axis=1, keepdims=True)
                e = jnp.exp(scores - smax)
                attn = e / jnp.sum(e, axis=1, keepdims=True)
                upd = jax.lax.dot_general(
                    attn, vh, (((1,), (0,)), ((), ())),
                    preferred_element_type=jnp.float32)
                vmean = jnp.mean(vh, axis=0, keepdims=True)
                ctx.append(vmean + jax.lax.dot_general(
                    oh, upd - vmean, (((0,), (0,)), ((), ())),
                    preferred_element_type=jnp.float32))
            o_ref[p] = jnp.concatenate(ctx, axis=1)

    full3 = pl.BlockSpec((_P, _L, 2 * _DH), lambda: (0, 0, 0))
    return pl.pallas_call(
        body,
        grid=(),
        in_specs=[
            pl.BlockSpec((_P, 2, _L), lambda: (0, 0, 0)),
            full3, full3, full3,
        ],
        out_specs=full3,
        out_shape=jax.ShapeDtypeStruct((_P, _L, 2 * _DH), jnp.float32),
        scratch_shapes=[
            pltpu.VMEM((_H, _U, _L), jnp.float32),
            pltpu.VMEM((_H, _U, _L), jnp.float32),
        ],
    )(meas, q3, k3, v3)


def _out_proj(ctx3, wo, bo):
    c = wo.shape[0]

    def body(x_ref, w_ref, b_ref, o_ref):
        x = jnp.concatenate([x_ref[p] for p in range(_P)], axis=1)
        acc = jax.lax.dot_general(x, w_ref[...], _DN,
                                  preferred_element_type=jnp.float32)
        o_ref[...] = acc + b_ref[...]

    return pl.pallas_call(
        body,
        grid=(_L // _BLK,),
        in_specs=[
            pl.BlockSpec((_P, _BLK, 2 * _DH), lambda i: (0, i, 0)),
            pl.BlockSpec((c, c), lambda i: (0, 0)),
            pl.BlockSpec((1, c), lambda i: (0, 0)),
        ],
        out_specs=pl.BlockSpec((_BLK, c), lambda i: (i, 0)),
        out_shape=jax.ShapeDtypeStruct((_L, c), jnp.float32),
    )(ctx3, wo, bo.reshape(1, c))


def kernel(query, key_in, value, Wq, bq, Wk, bk, Wv, bv, Wo, bo):
    lq, bq_dim, c = query.shape
    scale = (c // _H) ** -0.5
    x_q = query.reshape(lq, c)
    x_k = key_in.reshape(lq, c)
    x_v = value.reshape(lq, c)

    q3, k3, v3 = _qkv_proj(x_q, x_k, x_v, Wq, bq, Wk, bk, Wv, bv, scale)
    meas = _meas(q3, k3, _count_matrix())
    ctx3 = _attend(meas, q3, k3, v3)
    out = _out_proj(ctx3, Wo, bo)
    return out.reshape(bq_dim, lq, c)


# speedup vs baseline: 4.3509x; 4.3509x over previous
import math

import jax
import jax.numpy as jnp
import numpy as np
from jax.experimental import pallas as pl
from jax.experimental.pallas import tpu as pltpu

_L = 2048
_H = 16
_DH = 64
_P = _H // 2
_NODES = 8
_U = min(5 * int(math.ceil(math.log(_L + 1))), _L)

_IDX = np.asarray(jax.random.randint(jax.random.key(42), (_L, _U), 0, _L))
_CNT = np.zeros((_L, _L), np.float32)
np.add.at(_CNT, (np.arange(_L)[:, None], _IDX), 1.0)
_CNT_I8 = _CNT.astype(np.int8)

_BLK = 256
_DN = (((1,), (1,)), ((), ()))


def _qkv_proj(x_q, x_k, x_v, wq, bq, wk, bk, wv, bv, scale):
    c = x_q.shape[1]

    def body(xq_ref, xk_ref, xv_ref, wq_ref, bq_ref, wk_ref, bk_ref,
             wv_ref, bv_ref, q_ref, k_ref, v_ref):
        yq = (jax.lax.dot_general(xq_ref[...], wq_ref[...], _DN,
                                  preferred_element_type=jnp.float32)
              + bq_ref[...]) * scale
        yk = jax.lax.dot_general(xk_ref[...], wk_ref[...], _DN,
                                 preferred_element_type=jnp.float32) + bk_ref[...]
        yv = jax.lax.dot_general(xv_ref[...], wv_ref[...], _DN,
                                 preferred_element_type=jnp.float32) + bv_ref[...]
        for p in range(_P):
            sl = slice(p * 128, (p + 1) * 128)
            q_ref[p] = yq[:, sl]
            k_ref[p] = yk[:, sl]
            v_ref[p] = yv[:, sl]

    full_w = pl.BlockSpec((c, c), lambda i: (0, 0))
    full_b = pl.BlockSpec((1, c), lambda i: (0, 0))
    xspec = pl.BlockSpec((_BLK, c), lambda i: (i, 0))
    ospec = pl.BlockSpec((_P, _BLK, 2 * _DH), lambda i: (0, i, 0))
    oshape = jax.ShapeDtypeStruct((_P, _L, 2 * _DH), jnp.float32)
    return pl.pallas_call(
        body,
        grid=(_L // _BLK,),
        in_specs=[xspec, xspec, xspec, full_w, full_b, full_w, full_b,
                  full_w, full_b],
        out_specs=[ospec, ospec, ospec],
        out_shape=[oshape, oshape, oshape],
    )(x_q, x_k, x_v, wq, bq.reshape(1, c), wk, bk.reshape(1, c),
      wv, bv.reshape(1, c))


def _meas(q3, k3, cnt):
    def body(q_ref, k_ref, c_ref, o_ref):
        p = pl.program_id(1)
        cf = c_ref[...].astype(jnp.float32)
        kp = k_ref[p]
        for s in range(2):
            sl = slice(s * _DH, (s + 1) * _DH)
            qk = jax.lax.dot_general(
                q_ref[0][:, sl], kp[:, sl], _DN,
                preferred_element_type=jnp.float32)
            mx = jnp.max(jnp.where(cf > 0.0, qk, -jnp.inf), axis=1)
            sm = jnp.sum(qk * cf, axis=1) * (1.0 / _U)
            o_ref[0, s, :] = mx - sm

    return pl.pallas_call(
        body,
        grid=(_L // _BLK, _P),
        in_specs=[
            pl.BlockSpec((1, _BLK, 2 * _DH), lambda i, p: (p, i, 0)),
            pl.BlockSpec((_P, _L, 2 * _DH), lambda i, p: (0, 0, 0)),
            pl.BlockSpec((_BLK, _L), lambda i, p: (i, 0)),
        ],
        out_specs=pl.BlockSpec((1, 2, _BLK), lambda i, p: (p, 0, i)),
        out_shape=jax.ShapeDtypeStruct((_P, 2, _L), jnp.float32),
    )(q3, k3, cnt)


def _attend(meas, q3, k3, v3):
    def body(m_ref, q_ref, k_ref, v_ref, o_ref, oh_ref, ms_ref):
        iota_l = jax.lax.broadcasted_iota(jnp.int32, (1, _L), 1)
        mv = jnp.concatenate([m_ref[:, 0, :], m_ref[:, 1, :]], axis=0)
        for t in range(_U):
            m = jnp.max(mv, axis=1, keepdims=True)
            isel = jnp.min(jnp.where(mv == m, iota_l, _L), axis=1,
                           keepdims=True)
            hit = iota_l == isel
            oh_ref[:, t, :] = hit.astype(jnp.float32)
            ms_ref[:, t, :] = (
                (iota_l // _NODES) <= (isel // _NODES)).astype(jnp.float32)
            mv = jnp.where(hit, -jnp.inf, mv)

        for p in range(_P):
            ctx = []
            for s in range(2):
                r = p + s * _P
                sl = slice(s * _DH, (s + 1) * _DH)
                oh = oh_ref[r]
                qh = q_ref[p][:, sl]
                kh = k_ref[p][:, sl]
                vh = v_ref[p][:, sl]
                q_red = jax.lax.dot_general(
                    oh, qh, (((1,), (0,)), ((), ())),
                    preferred_element_type=jnp.float32)
                scores = jax.lax.dot_general(
                    q_red, kh, _DN,
                    preferred_element_type=jnp.float32)
                scores = jnp.where(ms_ref[r] > 0.5, scores, -jnp.inf)
                smax = jnp.max(scores, axis=1, keepdims=True)
                e = jnp.exp(scores - smax)
                attn = e / jnp.sum(e, axis=1, keepdims=True)
                upd = jax.lax.dot_general(
                    attn, vh, (((1,), (0,)), ((), ())),
                    preferred_element_type=jnp.float32)
                vmean = jnp.mean(vh, axis=0, keepdims=True)
                ctx.append(vmean + jax.lax.dot_general(
                    oh, upd - vmean, (((0,), (0,)), ((), ())),
                    preferred_element_type=jnp.float32))
            o_ref[p] = jnp.concatenate(ctx, axis=1)

    full3 = pl.BlockSpec((_P, _L, 2 * _DH), lambda: (0, 0, 0))
    return pl.pallas_call(
        body,
        grid=(),
        in_specs=[
            pl.BlockSpec((_P, 2, _L), lambda: (0, 0, 0)),
            full3, full3, full3,
        ],
        out_specs=full3,
        out_shape=jax.ShapeDtypeStruct((_P, _L, 2 * _DH), jnp.float32),
        scratch_shapes=[
            pltpu.VMEM((_H, _U, _L), jnp.float32),
            pltpu.VMEM((_H, _U, _L), jnp.float32),
        ],
    )(meas, q3, k3, v3)


def _out_proj(ctx3, wo, bo):
    c = wo.shape[0]

    def body(x_ref, w_ref, b_ref, o_ref):
        x = jnp.concatenate([x_ref[p] for p in range(_P)], axis=1)
        acc = jax.lax.dot_general(x, w_ref[...], _DN,
                                  preferred_element_type=jnp.float32)
        o_ref[...] = acc + b_ref[...]

    return pl.pallas_call(
        body,
        grid=(_L // _BLK,),
        in_specs=[
            pl.BlockSpec((_P, _BLK, 2 * _DH), lambda i: (0, i, 0)),
            pl.BlockSpec((c, c), lambda i: (0, 0)),
            pl.BlockSpec((1, c), lambda i: (0, 0)),
        ],
        out_specs=pl.BlockSpec((_BLK, c), lambda i: (i, 0)),
        out_shape=jax.ShapeDtypeStruct((_L, c), jnp.float32),
    )(ctx3, wo, bo.reshape(1, c))


def kernel(query, key_in, value, Wq, bq, Wk, bk, Wv, bv, Wo, bo):
    lq, bq_dim, c = query.shape
    scale = (c // _H) ** -0.5
    x_q = query.reshape(lq, c)
    x_k = key_in.reshape(lq, c)
    x_v = value.reshape(lq, c)

    q3, k3, v3 = _qkv_proj(x_q, x_k, x_v.astype(jnp.bfloat16), Wq, bq, Wk, bk,
                           Wv.astype(jnp.bfloat16), bv, scale)
    meas = _meas(q3, k3, jnp.asarray(_CNT_I8))
    ctx3 = _attend(meas, q3, k3, v3)
    out = _out_proj(ctx3.astype(jnp.bfloat16), Wo.astype(jnp.bfloat16), bo)
    return out.reshape(bq_dim, lq, c)


# speedup vs baseline: 4.6746x; 1.0744x over previous
import math

import jax
import jax.numpy as jnp
import numpy as np
from jax.experimental import pallas as pl
from jax.experimental.pallas import tpu as pltpu

_L = 2048
_H = 16
_DH = 64
_P = _H // 2
_NODES = 8
_U = min(5 * int(math.ceil(math.log(_L + 1))), _L)

_IDX = np.asarray(jax.random.randint(jax.random.key(42), (_L, _U), 0, _L))
_CNT = np.zeros((_L, _L), np.float32)
np.add.at(_CNT, (np.arange(_L)[:, None], _IDX), 1.0)
_CNT_I8 = _CNT.astype(np.int8)

_BLK = 256
_DN = (((1,), (1,)), ((), ()))


def _qkv_proj(x_q, x_k, x_v, wq, bq, wk, bk, wv, bv, scale):
    c = x_q.shape[1]

    def body(xq_ref, xk_ref, xv_ref, wq_ref, bq_ref, wk_ref, bk_ref,
             wv_ref, bv_ref, q_ref, k_ref, v_ref):
        yq = (jax.lax.dot_general(xq_ref[...], wq_ref[...], _DN,
                                  preferred_element_type=jnp.float32)
              + bq_ref[...]) * scale
        yk = jax.lax.dot_general(xk_ref[...], wk_ref[...], _DN,
                                 preferred_element_type=jnp.float32) + bk_ref[...]
        yv = jax.lax.dot_general(xv_ref[...], wv_ref[...], _DN,
                                 preferred_element_type=jnp.float32) + bv_ref[...]
        for p in range(_P):
            sl = slice(p * 128, (p + 1) * 128)
            q_ref[p] = yq[:, sl]
            k_ref[p] = yk[:, sl]
            v_ref[p] = yv[:, sl]

    full_w = pl.BlockSpec((c, c), lambda i: (0, 0))
    full_b = pl.BlockSpec((1, c), lambda i: (0, 0))
    xspec = pl.BlockSpec((_BLK, c), lambda i: (i, 0))
    ospec = pl.BlockSpec((_P, _BLK, 2 * _DH), lambda i: (0, i, 0))
    oshape = jax.ShapeDtypeStruct((_P, _L, 2 * _DH), jnp.float32)
    return pl.pallas_call(
        body,
        grid=(_L // _BLK,),
        in_specs=[xspec, xspec, xspec, full_w, full_b, full_w, full_b,
                  full_w, full_b],
        out_specs=[ospec, ospec, ospec],
        out_shape=[oshape, oshape, oshape],
    )(x_q, x_k, x_v, wq, bq.reshape(1, c), wk, bk.reshape(1, c),
      wv, bv.reshape(1, c))


def _meas(q3, k3, cnt):
    def body(q_ref, k_ref, c_ref, o_ref):
        p = pl.program_id(1)
        cf = c_ref[...].astype(jnp.float32)
        kp = k_ref[p]
        for s in range(2):
            sl = slice(s * _DH, (s + 1) * _DH)
            qk = jax.lax.dot_general(
                q_ref[0][:, sl], kp[:, sl], _DN,
                preferred_element_type=jnp.float32)
            mx = jnp.max(jnp.where(cf > 0.0, qk, -jnp.inf), axis=1)
            sm = jnp.sum(qk * cf, axis=1) * (1.0 / _U)
            o_ref[0, s, :] = mx - sm

    return pl.pallas_call(
        body,
        grid=(_L // _BLK, _P),
        in_specs=[
            pl.BlockSpec((1, _BLK, 2 * _DH), lambda i, p: (p, i, 0)),
            pl.BlockSpec((_P, _L, 2 * _DH), lambda i, p: (0, 0, 0)),
            pl.BlockSpec((_BLK, _L), lambda i, p: (i, 0)),
        ],
        out_specs=pl.BlockSpec((1, 2, _BLK), lambda i, p: (p, 0, i)),
        out_shape=jax.ShapeDtypeStruct((_P, 2, _L), jnp.float32),
    )(q3, k3, cnt)


def _attend(meas, q3, k3, v3, wo, bo):
    def body(m_ref, q_ref, k_ref, v_ref, wo_ref, bo_ref, o_ref,
             oh_ref, ctx_ref):
        iota_l = jax.lax.broadcasted_iota(jnp.int32, (1, _L), 1)
        iota_row = iota_l.astype(jnp.float32)
        mv = jnp.concatenate([m_ref[:, 0, :], m_ref[:, 1, :]], axis=0)
        for t in range(_U):
            m = jnp.max(mv, axis=1, keepdims=True)
            isel = jnp.min(jnp.where(mv == m, iota_l, _L), axis=1,
                           keepdims=True)
            hit = iota_l == isel
            oh_ref[:, t, :] = hit.astype(jnp.bfloat16)
            mv = jnp.where(hit, -jnp.inf, mv)

        for p in range(_P):
            ctx = []
            for s in range(2):
                r = p + s * _P
                sl = slice(s * _DH, (s + 1) * _DH)
                oh = oh_ref[r].astype(jnp.float32)
                qh = q_ref[p][:, sl]
                kh = k_ref[p][:, sl]
                vh = v_ref[p][:, sl]
                q_red = jax.lax.dot_general(
                    oh, qh, (((1,), (0,)), ((), ())),
                    preferred_element_type=jnp.float32)
                scores = jax.lax.dot_general(
                    q_red, kh, _DN,
                    preferred_element_type=jnp.float32)
                iselv = jnp.max(oh * iota_row, axis=1, keepdims=True)
                smask = (iota_l // _NODES) <= (iselv.astype(jnp.int32)
                                               // _NODES)
                scores = jnp.where(smask, scores, -jnp.inf)
                smax = jnp.max(scores, axis=1, keepdims=True)
                e = jnp.exp(scores - smax)
                attn = e / jnp.sum(e, axis=1, keepdims=True)
                upd = jax.lax.dot_general(
                    attn, vh, (((1,), (0,)), ((), ())),
                    preferred_element_type=jnp.float32)
                vmean = jnp.mean(vh, axis=0, keepdims=True)
                ctx.append(vmean + jax.lax.dot_general(
                    oh, upd - vmean, (((0,), (0,)), ((), ())),
                    preferred_element_type=jnp.float32))
            ctx_ref[:, p * 2 * _DH:(p + 1) * 2 * _DH] = jnp.concatenate(
                ctx, axis=1).astype(jnp.bfloat16)

        acc = jax.lax.dot_general(ctx_ref[...], wo_ref[...], _DN,
                                  preferred_element_type=jnp.float32)
        o_ref[...] = acc + bo_ref[...]

    c = _H * _DH
    full3 = pl.BlockSpec((_P, _L, 2 * _DH), lambda: (0, 0, 0))
    return pl.pallas_call(
        body,
        grid=(),
        in_specs=[
            pl.BlockSpec((_P, 2, _L), lambda: (0, 0, 0)),
            full3, full3, full3,
            pl.BlockSpec((c, c), lambda: (0, 0)),
            pl.BlockSpec((1, c), lambda: (0, 0)),
        ],
        out_specs=pl.BlockSpec((_L, c), lambda: (0, 0)),
        out_shape=jax.ShapeDtypeStruct((_L, c), jnp.float32),
        scratch_shapes=[
            pltpu.VMEM((_H, _U, _L), jnp.bfloat16),
            pltpu.VMEM((_L, _H * _DH), jnp.bfloat16),
        ],
    )(meas, q3, k3, v3, wo, bo)


def kernel(query, key_in, value, Wq, bq, Wk, bk, Wv, bv, Wo, bo):
    lq, bq_dim, c = query.shape
    scale = (c // _H) ** -0.5
    x_q = query.reshape(lq, c)
    x_k = key_in.reshape(lq, c)
    x_v = value.reshape(lq, c)

    q3, k3, v3 = _qkv_proj(x_q, x_k, x_v.astype(jnp.bfloat16), Wq, bq, Wk, bk,
                           Wv.astype(jnp.bfloat16), bv, scale)
    meas = _meas(q3, k3, jnp.asarray(_CNT_I8))
    out = _attend(meas, q3, k3, v3, Wo.astype(jnp.bfloat16),
                  bo.reshape(1, c))
    return out.reshape(bq_dim, lq, c)
